# Initial kernel scaffold; baseline (speedup 1.0000x reference)
import jax
import jax.numpy as jnp
from jax.experimental import pallas as pl


def kernel(ids, boxes):
    raise NotImplementedError("write your pallas kernel here")


# speedup vs baseline: 1.7180x; 1.7180x over previous
import functools

import jax
import jax.numpy as jnp
from jax import lax
from jax.experimental import pallas as pl
from jax.experimental.pallas import tpu as pltpu
from jax.experimental.pallas import tpu_sc as plsc

L = 16
NC = 2
NS = 16
NW = NC * NS
IDX_CHUNK = 128


def _lane_shuffle(x, idx):
    dnums = lax.GatherDimensionNumbers(
        offset_dims=(), collapsed_slice_dims=(0,), start_index_map=(0,))
    return lax.gather(
        x, idx[:, None], dnums, slice_sizes=(1,),
        mode=lax.GatherScatterMode.PROMISE_IN_BOUNDS)


@functools.partial(jax.jit, static_argnums=(3,))
def _cond_probs_sc(ids_a, ids_b, table, dim):
    batch = ids_a.shape[0]
    rw = 2 * dim
    b_per_w = batch // NW
    n_chunks = b_per_w // IDX_CHUNK
    n_blocks = b_per_w // L

    mesh = plsc.VectorSubcoreMesh(core_axis_name="c", subcore_axis_name="s")

    @functools.partial(
        pl.kernel,
        out_type=(
            jax.ShapeDtypeStruct((batch, rw), jnp.float32),
            jax.ShapeDtypeStruct((batch, rw), jnp.float32),
            jax.ShapeDtypeStruct((batch,), jnp.float32),
        ),
        mesh=mesh,
        scratch_types=[
            pltpu.VMEM((n_chunks, IDX_CHUNK), jnp.int32),
            pltpu.VMEM((n_chunks, IDX_CHUNK), jnp.int32),
            pltpu.VMEM((b_per_w, rw), jnp.float32),
            pltpu.VMEM((b_per_w, rw), jnp.float32),
            pltpu.VMEM((b_per_w,), jnp.float32),
            pltpu.SemaphoreType.DMA,
        ],
        compiler_params=pltpu.CompilerParams(use_tc_tiling_on_sc=False),
    )
    def sc_kernel(ids_a_hbm, ids_b_hbm, table_hbm, rows_a_out, rows_b_out,
                  p_out, idx_a_v, idx_b_v, rows_a_v, rows_b_v, p_v, sem):
        wid = lax.axis_index("s") * NC + lax.axis_index("c")
        base = wid * b_per_w
        copies = []
        for j in range(n_chunks):
            pltpu.sync_copy(
                ids_a_hbm.at[pl.ds(base + j * IDX_CHUNK, IDX_CHUNK)],
                idx_a_v.at[j],
            )
            pltpu.sync_copy(
                ids_b_hbm.at[pl.ds(base + j * IDX_CHUNK, IDX_CHUNK)],
                idx_b_v.at[j],
            )
            dst_a = rows_a_v.at[pl.ds(j * IDX_CHUNK, IDX_CHUNK)]
            dst_b = rows_b_v.at[pl.ds(j * IDX_CHUNK, IDX_CHUNK)]
            copies.append(pltpu.async_copy(table_hbm.at[idx_a_v.at[j]], dst_a, sem))
            copies.append(pltpu.async_copy(table_hbm.at[idx_b_v.at[j]], dst_b, sem))
        for cp in copies:
            cp.wait()

        lane = lax.iota(jnp.int32, L)

        def block(b, carry):
            base_i = pl.multiple_of(b * L, L)
            acc = jnp.zeros((L,), jnp.float32)
            for t in range(L):
                i = base_i + t
                za0 = rows_a_v[i, pl.ds(0, L)]
                za1 = rows_a_v[i, pl.ds(L, L)]
                ha0 = rows_a_v[i, pl.ds(2 * L, L)]
                ha1 = rows_a_v[i, pl.ds(3 * L, L)]
                zb0 = rows_b_v[i, pl.ds(0, L)]
                zb1 = rows_b_v[i, pl.ds(L, L)]
                hb0 = rows_b_v[i, pl.ds(2 * L, L)]
                hb1 = rows_b_v[i, pl.ds(3 * L, L)]
                w0 = jnp.maximum(
                    jnp.minimum(ha0, hb0) - jnp.maximum(za0, zb0), 0.0)
                w1 = jnp.maximum(
                    jnp.minimum(ha1, hb1) - jnp.maximum(za1, zb1), 0.0)
                s0 = jnp.maximum(ha0 - za0, 0.0)
                s1 = jnp.maximum(ha1 - za1, 0.0)
                q = (w0 * w1) / (s0 * s1)
                for sh in (8, 4, 2, 1):
                    q = q * _lane_shuffle(q, lane ^ sh)
                acc = jnp.where(lane == t, q, acc)
            p_v[pl.ds(pl.multiple_of(b * L, L), L)] = acc
            return carry

        lax.fori_loop(0, n_blocks, block, 0)

        pltpu.sync_copy(rows_a_v, rows_a_out.at[pl.ds(base, b_per_w)])
        pltpu.sync_copy(rows_b_v, rows_b_out.at[pl.ds(base, b_per_w)])
        pltpu.sync_copy(p_v, p_out.at[pl.ds(base, b_per_w)])

    return sc_kernel(ids_a, ids_b, table)


def kernel(ids, boxes):
    num_models, num_boxes, _, dim = boxes.shape
    batch = ids.shape[1]
    table = boxes.reshape(num_boxes, 2 * dim)
    rows_a, rows_b, p = _cond_probs_sc(ids[0], ids[1], table, dim)
    a = rows_a.reshape(num_models, batch, 2, dim)
    b = rows_b.reshape(num_models, batch, 2, dim)
    return (p.reshape(num_models, batch), a, b)
